# Initial kernel scaffold; baseline (speedup 1.0000x reference)
import jax
import jax.numpy as jnp
from jax.experimental import pallas as pl


def kernel(x, x_n, y_n):
    raise NotImplementedError("write your pallas kernel here")


# speedup vs baseline: 12.9118x; 12.9118x over previous
import jax
import jax.numpy as jnp
from jax.experimental import pallas as pl
from jax.experimental.pallas import tpu as pltpu

_K = 64
_C = 100
_CPAD = 128
_QB = 128


def _nona_body(x_ref, xnt_ref, y_ref, out_ref, d2_ref):
    xb = x_ref[...]
    xnt = xnt_ref[...]
    n = xnt.shape[1]

    x2 = jnp.sum(xb * xb, axis=1, keepdims=True)
    xn2 = jnp.sum(xnt * xnt, axis=0, keepdims=True)
    dot = jax.lax.dot_general(
        xb, xnt, (((1,), (0,)), ((), ())),
        preferred_element_type=jnp.float32,
        precision=jax.lax.Precision.DEFAULT)
    d2_ref[...] = jnp.maximum(x2 + xn2 - 2.0 * dot, 0.0)

    d2min = jnp.min(d2_ref[...], axis=1, keepdims=True)
    d2max = jnp.max(d2_ref[...], axis=1, keepdims=True)
    lo = jax.lax.bitcast_convert_type(d2min, jnp.int32)
    hi = jax.lax.bitcast_convert_type(d2max, jnp.int32)

    def step(_, carry):
        lo, hi = carry
        mid = lo + ((hi - lo) >> 1)
        midf = jax.lax.bitcast_convert_type(mid, jnp.float32)
        cnt = jnp.sum((d2_ref[...] <= midf).astype(jnp.float32),
                      axis=1, keepdims=True)
        pred = cnt >= _K
        return jnp.where(pred, lo, mid + 1), jnp.where(pred, mid, hi)

    lo, hi = jax.lax.fori_loop(0, 31, step, (lo, hi))
    t = jax.lax.bitcast_convert_type(hi, jnp.float32)

    d2v = d2_ref[...]
    lt = d2v < t
    eq = d2v == t
    cnt_lt = jnp.sum(lt.astype(jnp.float32), axis=1, keepdims=True)
    cnt_eq = jnp.sum(eq.astype(jnp.float32), axis=1, keepdims=True)
    scale = (_K - cnt_lt) / cnt_eq
    m = jnp.sqrt(d2min)
    wfull = jnp.exp(m - jnp.sqrt(d2v))
    d2_ref[...] = jnp.where(lt, wfull, jnp.where(eq, wfull * scale, 0.0))

    yv = y_ref[...]
    cls = jax.lax.broadcasted_iota(jnp.int32, (n, _CPAD), 1)
    yoh = (yv == cls).astype(jnp.float32)
    numer = jax.lax.dot_general(
        d2_ref[...], yoh, (((1,), (0,)), ((), ())),
        preferred_element_type=jnp.float32,
        precision=jax.lax.Precision.HIGHEST)
    z = jnp.sum(numer, axis=1, keepdims=True)
    out_ref[...] = jnp.clip(numer / z, 0.0, 1.0)


def kernel(x, x_n, y_n):
    q, d = x.shape
    n = x_n.shape[0]
    xnt = x_n.T
    y2 = y_n.reshape(n, 1)
    out = pl.pallas_call(
        _nona_body,
        grid=(q // _QB,),
        in_specs=[
            pl.BlockSpec((_QB, d), lambda i: (i, 0)),
            pl.BlockSpec((d, n), lambda i: (0, 0)),
            pl.BlockSpec((n, 1), lambda i: (0, 0)),
        ],
        out_specs=pl.BlockSpec((_QB, _CPAD), lambda i: (i, 0)),
        out_shape=jax.ShapeDtypeStruct((q, _CPAD), jnp.float32),
        scratch_shapes=[pltpu.VMEM((_QB, n), jnp.float32)],
        compiler_params=pltpu.CompilerParams(
            dimension_semantics=("parallel",)),
    )(x, xnt, y2)
    return out[:, :_C]


# speedup vs baseline: 16.2148x; 1.2558x over previous
import jax
import jax.numpy as jnp
from jax.experimental import pallas as pl
from jax.experimental.pallas import tpu as pltpu

_K = 64
_C = 100
_CPAD = 128
_QB = 256


def _nona_body(x_ref, xnt_ref, y_ref, out_ref, d2_ref):
    xb = x_ref[...]
    xnt = xnt_ref[...]
    n = xnt.shape[1]

    x2 = jnp.sum(xb * xb, axis=1, keepdims=True)
    xn2 = jnp.sum(xnt * xnt, axis=0, keepdims=True)
    dot = jax.lax.dot_general(
        xb, xnt, (((1,), (0,)), ((), ())),
        preferred_element_type=jnp.float32,
        precision=jax.lax.Precision.DEFAULT)
    d2_ref[...] = jnp.maximum(x2 + xn2 - 2.0 * dot, 0.0)

    d2min = jnp.min(d2_ref[...], axis=1, keepdims=True)
    d2max = jnp.max(d2_ref[...], axis=1, keepdims=True)
    lo = jax.lax.bitcast_convert_type(d2min, jnp.int32)
    hi = jax.lax.bitcast_convert_type(d2max, jnp.int32)

    def step(_, carry):
        lo, hi = carry
        mid = lo + ((hi - lo) >> 1)
        midf = jax.lax.bitcast_convert_type(mid, jnp.float32)
        cnt = jnp.sum((d2_ref[...] <= midf).astype(jnp.float32),
                      axis=1, keepdims=True)
        pred = cnt >= _K
        return jnp.where(pred, lo, mid + 1), jnp.where(pred, mid, hi)

    lo, hi = jax.lax.fori_loop(0, 31, step, (lo, hi))
    t = jax.lax.bitcast_convert_type(hi, jnp.float32)

    d2v = d2_ref[...]
    lt = d2v < t
    eq = d2v == t
    cnt_lt = jnp.sum(lt.astype(jnp.float32), axis=1, keepdims=True)
    cnt_eq = jnp.sum(eq.astype(jnp.float32), axis=1, keepdims=True)
    scale = (_K - cnt_lt) / cnt_eq
    m = jnp.sqrt(d2min)
    wfull = jnp.exp(m - jnp.sqrt(d2v))
    d2_ref[...] = jnp.where(lt, wfull, jnp.where(eq, wfull * scale, 0.0))

    yv = y_ref[...]
    cls = jax.lax.broadcasted_iota(jnp.int8, (n, _CPAD), 1)
    yoh = (yv == cls).astype(jnp.float32)
    numer = jax.lax.dot_general(
        d2_ref[...], yoh, (((1,), (0,)), ((), ())),
        preferred_element_type=jnp.float32,
        precision=jax.lax.Precision.DEFAULT)
    z = jnp.sum(numer, axis=1, keepdims=True)
    out_ref[...] = jnp.clip(numer / z, 0.0, 1.0)


def kernel(x, x_n, y_n):
    q, d = x.shape
    n = x_n.shape[0]
    xnt = x_n.T
    y2 = y_n.astype(jnp.int8).reshape(n, 1)
    out = pl.pallas_call(
        _nona_body,
        grid=(q // _QB,),
        in_specs=[
            pl.BlockSpec((_QB, d), lambda i: (i, 0)),
            pl.BlockSpec((d, n), lambda i: (0, 0)),
            pl.BlockSpec((n, 1), lambda i: (0, 0)),
        ],
        out_specs=pl.BlockSpec((_QB, _CPAD), lambda i: (i, 0)),
        out_shape=jax.ShapeDtypeStruct((q, _CPAD), jnp.float32),
        scratch_shapes=[pltpu.VMEM((_QB, n), jnp.float32)],
        compiler_params=pltpu.CompilerParams(
            dimension_semantics=("parallel",)),
    )(x, xnt, y2)
    return out[:, :_C]


# speedup vs baseline: 18.9186x; 1.1667x over previous
import jax
import jax.numpy as jnp
from jax.experimental import pallas as pl
from jax.experimental.pallas import tpu as pltpu

_K = 64
_C = 100
_CPAD = 128
_QB = 256


def _nona_body(x_ref, xnt_ref, y_ref, out_ref, d2_ref):
    xb = x_ref[...]
    xnt = xnt_ref[...]
    n = xnt.shape[1]

    x2 = jnp.sum(xb * xb, axis=1, keepdims=True)
    xn2 = jnp.sum(xnt * xnt, axis=0, keepdims=True)
    dot = jax.lax.dot_general(
        xb, xnt, (((1,), (0,)), ((), ())),
        preferred_element_type=jnp.float32,
        precision=jax.lax.Precision.DEFAULT)
    d2_ref[...] = jnp.maximum(x2 + xn2 - 2.0 * dot, 0.0)

    d2v0 = d2_ref[...]
    ng = n // 128
    gmin = d2v0[:, 0:128]
    for g in range(1, ng):
        gmin = jnp.minimum(gmin, d2v0[:, g * 128:(g + 1) * 128])
    d2min = jnp.min(gmin, axis=1, keepdims=True)

    glo = jax.lax.bitcast_convert_type(d2min, jnp.int32)
    ghi = jax.lax.bitcast_convert_type(
        jnp.max(gmin, axis=1, keepdims=True), jnp.int32)

    def gstep(_, carry):
        lo, hi = carry
        mid = lo + ((hi - lo) >> 1)
        midf = jax.lax.bitcast_convert_type(mid, jnp.float32)
        cnt = jnp.sum((gmin <= midf).astype(jnp.float32),
                      axis=1, keepdims=True)
        pred = cnt >= _K
        return jnp.where(pred, lo, mid + 1), jnp.where(pred, mid, hi)

    _, ghi = jax.lax.fori_loop(0, 31, gstep, (glo, ghi))

    lo = jax.lax.bitcast_convert_type(d2min, jnp.int32)
    hi = ghi

    def cond(carry):
        lo, hi = carry
        return jnp.any(lo < hi)

    def step(carry):
        lo, hi = carry
        mid = lo + ((hi - lo) >> 1)
        midf = jax.lax.bitcast_convert_type(mid, jnp.float32)
        cnt = jnp.sum((d2_ref[...] <= midf).astype(jnp.float32),
                      axis=1, keepdims=True)
        pred = cnt >= _K
        return jnp.where(pred, lo, mid + 1), jnp.where(pred, mid, hi)

    lo, hi = jax.lax.while_loop(cond, step, (lo, hi))
    t = jax.lax.bitcast_convert_type(hi, jnp.float32)

    d2v = d2_ref[...]
    lt = d2v < t
    eq = d2v == t
    cnt_lt = jnp.sum(lt.astype(jnp.float32), axis=1, keepdims=True)
    cnt_eq = jnp.sum(eq.astype(jnp.float32), axis=1, keepdims=True)
    scale = (_K - cnt_lt) / cnt_eq
    m = jnp.sqrt(d2min)
    wfull = jnp.exp(m - jnp.sqrt(d2v))
    d2_ref[...] = jnp.where(lt, wfull, jnp.where(eq, wfull * scale, 0.0))

    yv = y_ref[...]
    cls = jax.lax.broadcasted_iota(jnp.int8, (n, _CPAD), 1)
    yoh = (yv == cls).astype(jnp.float32)
    numer = jax.lax.dot_general(
        d2_ref[...], yoh, (((1,), (0,)), ((), ())),
        preferred_element_type=jnp.float32,
        precision=jax.lax.Precision.DEFAULT)
    z = jnp.sum(numer, axis=1, keepdims=True)
    out_ref[...] = jnp.clip(numer / z, 0.0, 1.0)


def kernel(x, x_n, y_n):
    q, d = x.shape
    n = x_n.shape[0]
    xnt = x_n.T
    y2 = y_n.astype(jnp.int8).reshape(n, 1)
    out = pl.pallas_call(
        _nona_body,
        grid=(q // _QB,),
        in_specs=[
            pl.BlockSpec((_QB, d), lambda i: (i, 0)),
            pl.BlockSpec((d, n), lambda i: (0, 0)),
            pl.BlockSpec((n, 1), lambda i: (0, 0)),
        ],
        out_specs=pl.BlockSpec((_QB, _CPAD), lambda i: (i, 0)),
        out_shape=jax.ShapeDtypeStruct((q, _CPAD), jnp.float32),
        scratch_shapes=[pltpu.VMEM((_QB, n), jnp.float32)],
        compiler_params=pltpu.CompilerParams(
            dimension_semantics=("parallel",)),
    )(x, xnt, y2)
    return out[:, :_C]


# speedup vs baseline: 24.5517x; 1.2978x over previous
import jax
import jax.numpy as jnp
from jax.experimental import pallas as pl
from jax.experimental.pallas import tpu as pltpu

_K = 64
_C = 100
_CPAD = 128
_QB = 256


def _nona_body(x_ref, xnt_ref, y_ref, out_ref, d2_ref):
    xb = x_ref[...]
    xnt = xnt_ref[...]
    n = xnt.shape[1]

    x2 = jnp.sum(xb * xb, axis=1, keepdims=True)
    xn2 = jnp.sum(xnt * xnt, axis=0, keepdims=True)
    dot = jax.lax.dot_general(
        xb, xnt, (((1,), (0,)), ((), ())),
        preferred_element_type=jnp.float32,
        precision=jax.lax.Precision.DEFAULT)
    d2_ref[...] = jnp.maximum(x2 + xn2 - 2.0 * dot, 0.0)

    d2v0 = d2_ref[...]
    ng = n // 128
    gmin = d2v0[:, 0:128]
    for g in range(1, ng):
        gmin = jnp.minimum(gmin, d2v0[:, g * 128:(g + 1) * 128])
    d2min = jnp.min(gmin, axis=1, keepdims=True)

    glo = jax.lax.bitcast_convert_type(d2min, jnp.int32)
    ghi = jax.lax.bitcast_convert_type(
        jnp.max(gmin, axis=1, keepdims=True), jnp.int32)

    def gstep(_, carry):
        lo, hi = carry
        mid = lo + ((hi - lo) >> 1)
        midf = jax.lax.bitcast_convert_type(mid, jnp.float32)
        cnt = jnp.sum((gmin <= midf).astype(jnp.float32),
                      axis=1, keepdims=True)
        pred = cnt >= _K
        return jnp.where(pred, lo, mid + 1), jnp.where(pred, mid, hi)

    _, ghi = jax.lax.fori_loop(0, 12, gstep, (glo, ghi))

    lo = jax.lax.bitcast_convert_type(d2min, jnp.int32)
    hi = ghi

    def cond(carry):
        lo, hi = carry
        return jnp.any((hi - lo) > 127)

    def step(carry):
        lo, hi = carry
        mid = lo + ((hi - lo) >> 1)
        midf = jax.lax.bitcast_convert_type(mid, jnp.float32)
        cnt = jnp.sum((d2_ref[...] <= midf).astype(jnp.float32),
                      axis=1, keepdims=True)
        pred = cnt >= _K
        return jnp.where(pred, lo, mid + 1), jnp.where(pred, mid, hi)

    lo, hi = jax.lax.while_loop(cond, step, (lo, hi))
    t_lo = jax.lax.bitcast_convert_type(lo, jnp.float32)
    t_hi = jax.lax.bitcast_convert_type(hi, jnp.float32)

    d2v = d2_ref[...]
    lt = d2v < t_lo
    le = d2v <= t_hi
    cnt_lt = jnp.sum(lt.astype(jnp.float32), axis=1, keepdims=True)
    cnt_le = jnp.sum(le.astype(jnp.float32), axis=1, keepdims=True)
    scale = (_K - cnt_lt) / (cnt_le - cnt_lt)
    m = jnp.sqrt(d2min)
    wfull = jnp.exp(m - jnp.sqrt(d2v))
    d2_ref[...] = jnp.where(lt, wfull, jnp.where(le, wfull * scale, 0.0))

    yv = y_ref[...]
    cls = jax.lax.broadcasted_iota(jnp.int8, (n, _CPAD), 1)
    yoh = (yv == cls).astype(jnp.float32)
    numer = jax.lax.dot_general(
        d2_ref[...], yoh, (((1,), (0,)), ((), ())),
        preferred_element_type=jnp.float32,
        precision=jax.lax.Precision.DEFAULT)
    z = jnp.sum(numer, axis=1, keepdims=True)
    out_ref[...] = jnp.clip(numer / z, 0.0, 1.0)


def kernel(x, x_n, y_n):
    q, d = x.shape
    n = x_n.shape[0]
    xnt = x_n.T
    y2 = y_n.astype(jnp.int8).reshape(n, 1)
    out = pl.pallas_call(
        _nona_body,
        grid=(q // _QB,),
        in_specs=[
            pl.BlockSpec((_QB, d), lambda i: (i, 0)),
            pl.BlockSpec((d, n), lambda i: (0, 0)),
            pl.BlockSpec((n, 1), lambda i: (0, 0)),
        ],
        out_specs=pl.BlockSpec((_QB, _CPAD), lambda i: (i, 0)),
        out_shape=jax.ShapeDtypeStruct((q, _CPAD), jnp.float32),
        scratch_shapes=[pltpu.VMEM((_QB, n), jnp.float32)],
        compiler_params=pltpu.CompilerParams(
            dimension_semantics=("parallel",)),
    )(x, xnt, y2)
    return out[:, :_C]
